# baseline (speedup 1.0000x reference)
import jax
import jax.numpy as jnp
from jax import lax
from jax.experimental import pallas as pl
from jax.experimental.pallas import tpu as pltpu
from jax.experimental.pallas import tpu_sc as plsc

_GRID = 8
_DIM = 3
_A = 16
_S = _GRID ** 3
_NW = 32
_CH = 3136
_PPAD = _NW * _CH
_IMAX = 0x7FFFFFFF


def _grid_init_kernel(pts_t_ref, px_t_ref, mn_ref, mx_ref):
    mn = jnp.min(pts_t_ref[...], axis=1, keepdims=True)
    mx = jnp.max(pts_t_ref[...], axis=1, keepdims=True)
    mn_ref[...] = mn
    mx_ref[...] = mx
    ge = (mx - mn) / jnp.float32(_GRID) * jnp.float32(0.5)
    r = lax.broadcasted_iota(jnp.int32, (_DIM, _S), 0)
    s = lax.broadcasted_iota(jnp.int32, (_DIM, _S), 1)
    mesh = jnp.where(r == 0, s // (_GRID * _GRID),
                     jnp.where(r == 1, (s // _GRID) % _GRID, s % _GRID))
    mesh_ph = mesh.astype(jnp.float32) + jnp.float32(0.5)
    px_t_ref[...] = mesh_ph * ge * jnp.float32(2.0) + mn


def _sc_knn_body(xh, yh, zh, ph, out_h, xv, yv, zv, pv, ov):
    f32 = jnp.float32
    i32 = jnp.int32
    wid = lax.axis_index("c") * 16 + lax.axis_index("s")
    base = wid * _CH
    pltpu.sync_copy(xh.at[pl.ds(base, _CH)], xv)
    pltpu.sync_copy(yh.at[pl.ds(base, _CH)], yv)
    pltpu.sync_copy(zh.at[pl.ds(base, _CH)], zv)
    pltpu.sync_copy(ph, pv)
    pvec = pv[...]
    mnx, mny, mnz = pvec[0], pvec[1], pvec[2]
    gex, gey, gez = pvec[3], pvec[4], pvec[5]
    inv_sx, inv_sy, inv_sz = pvec[6], pvec[7], pvec[8]
    iotav = jnp.arange(16, dtype=i32)
    oy = iotav // 4
    oz = iotav % 4
    inf_f = f32(jnp.inf)

    def center(idx_f, ge, mn):
        return (idx_f + f32(0.5)) * ge * f32(2.0) + mn

    def group_body(g, carry):
        g16 = g * 16
        xg = xv[pl.ds(g16, 16)]
        yg = yv[pl.ds(g16, 16)]
        zg = zv[pl.ds(g16, 16)]
        for j in range(16):
            _one_point(g16 + j, xg[j], yg[j], zg[j])
        return carry

    def _one_point(row, xj, yj, zj):
        bx = jnp.clip(((xj - mnx) * inv_sx - f32(0.5)).astype(i32) - 1, 0, 4)
        by = jnp.clip(((yj - mny) * inv_sy - f32(0.5)).astype(i32) - 1, 0, 4)
        bz = jnp.clip(((zj - mnz) * inv_sz - f32(0.5)).astype(i32) - 1, 0, 4)
        ptsq = xj * xj + yj * yj + zj * zj
        cyv = center((by + oy).astype(f32), gey, mny)
        czv = center((bz + oz).astype(f32), gez, mnz)
        gyz = (by + oy) * 8 + (bz + oz)
        pyz = cyv * cyv + czv * czv
        dyz = yj * cyv + zj * czv
        keys = []
        for r in range(4):
            cxs = center((bx + r).astype(f32), gex, mnx)
            d2 = (ptsq + (cxs * cxs + pyz)) - f32(2.0) * (xj * cxs + dyz)
            gi = (bx + r) * 64 + gyz
            key = (lax.bitcast_convert_type(d2, i32) & i32(-512)) | gi
            keys.append(jnp.sort(key))
        m01 = jnp.sort(jnp.minimum(keys[0], lax.rev(keys[1], (0,))))
        m23 = jnp.sort(jnp.minimum(keys[2], lax.rev(keys[3], (0,))))
        top = jnp.sort(jnp.minimum(m01, lax.rev(m23, (0,))))
        def sqd(pj, b, ge, mn):
            dd = pj - center(b.astype(f32), ge, mn)
            return dd * dd

        exl = jnp.where(bx > 0, sqd(xj, bx - 1, gex, mnx), inf_f)
        exh = jnp.where(bx < 4, sqd(xj, bx + 4, gex, mnx), inf_f)
        eyl = jnp.where(by > 0, sqd(yj, by - 1, gey, mny), inf_f)
        eyh = jnp.where(by < 4, sqd(yj, by + 4, gey, mny), inf_f)
        ezl = jnp.where(bz > 0, sqd(zj, bz - 1, gez, mnz), inf_f)
        ezh = jnp.where(bz < 4, sqd(zj, bz + 4, gez, mnz), inf_f)
        e = jnp.minimum(jnp.minimum(jnp.minimum(exl, exh), jnp.minimum(eyl, eyh)),
                        jnp.minimum(ezl, ezh))
        ebits = lax.bitcast_convert_type(jnp.broadcast_to(e, (16,)), i32) & i32(-512)
        safe = jnp.all(top < ebits)

        def full_scan(_):
            def fb(r, run):
                s = iotav + r * 16
                cxv = center((s // 64).astype(f32), gex, mnx)
                cyv2 = center(((s // 8) % 8).astype(f32), gey, mny)
                czv2 = center((s % 8).astype(f32), gez, mnz)
                pxsq = cxv * cxv + cyv2 * cyv2 + czv2 * czv2
                dot = xj * cxv + yj * cyv2 + zj * czv2
                d2 = (ptsq + pxsq) - f32(2.0) * dot
                key = (lax.bitcast_convert_type(d2, i32) & i32(-512)) | s
                return jnp.sort(jnp.minimum(run, lax.rev(jnp.sort(key), (0,))))
            return lax.fori_loop(0, 32, fb, jnp.full((16,), i32(_IMAX)))

        top = lax.cond(safe, lambda t: t, full_scan, top)
        ov[pl.ds(row * 16, 16)] = top & i32(_S - 1)

    lax.fori_loop(0, _CH // 16, group_body, 0)
    pltpu.sync_copy(ov, out_h.at[pl.ds(base * _A, _CH * _A)])


def kernel(point_pos):
    p = point_pos.shape[0]
    pts = jnp.pad(point_pos, ((0, _PPAD - p), (0, 0)), mode="edge")
    pts_t = pts.T

    px_t, mn, mx = pl.pallas_call(
        _grid_init_kernel,
        out_shape=(
            jax.ShapeDtypeStruct((_DIM, _S), jnp.float32),
            jax.ShapeDtypeStruct((_DIM, 1), jnp.float32),
            jax.ShapeDtypeStruct((_DIM, 1), jnp.float32),
        ),
    )(pts_t)

    mn1 = mn[:, 0]
    ge = (mx[:, 0] - mn1) / jnp.float32(_GRID) * jnp.float32(0.5)
    inv_step = jnp.float32(1.0) / (ge * jnp.float32(2.0))
    params = jnp.concatenate([mn1, ge, inv_step, jnp.zeros((7,), jnp.float32)])

    mesh = plsc.VectorSubcoreMesh(core_axis_name="c", subcore_axis_name="s")
    idx = pl.kernel(
        _sc_knn_body,
        out_type=jax.ShapeDtypeStruct((_PPAD * _A,), jnp.int32),
        mesh=mesh,
        compiler_params=pltpu.CompilerParams(needs_layout_passes=False),
        scratch_types=[
            pltpu.VMEM((_CH,), jnp.float32),
            pltpu.VMEM((_CH,), jnp.float32),
            pltpu.VMEM((_CH,), jnp.float32),
            pltpu.VMEM((16,), jnp.float32),
            pltpu.VMEM((_CH * _A,), jnp.int32),
        ],
    )(pts_t[0], pts_t[1], pts_t[2], params)
    idx = idx.reshape(_PPAD, _A)

    px_pos = px_t.T
    pt_ids = jnp.repeat(jnp.arange(p, dtype=jnp.int32), _A)
    px_ids = idx[:p].reshape(-1)
    assoc = jnp.stack([pt_ids, px_ids], axis=-1)
    return px_pos, assoc


# speedup vs baseline: 5.6364x; 5.6364x over previous
import jax
import jax.numpy as jnp
from jax import lax
from jax.experimental import pallas as pl
from jax.experimental.pallas import tpu as pltpu
from jax.experimental.pallas import tpu_sc as plsc

_GRID = 8
_DIM = 3
_A = 16
_S = _GRID ** 3
_NW = 32
_CH = 3136
_PPAD = _NW * _CH
_IMAX = 0x7FFFFFFF


def _grid_init_kernel(pts_t_ref, px_t_ref, mn_ref, mx_ref):
    mn = jnp.min(pts_t_ref[...], axis=1, keepdims=True)
    mx = jnp.max(pts_t_ref[...], axis=1, keepdims=True)
    mn_ref[...] = mn
    mx_ref[...] = mx
    ge = (mx - mn) / jnp.float32(_GRID) * jnp.float32(0.5)
    r = lax.broadcasted_iota(jnp.int32, (_DIM, _S), 0)
    s = lax.broadcasted_iota(jnp.int32, (_DIM, _S), 1)
    mesh = jnp.where(r == 0, s // (_GRID * _GRID),
                     jnp.where(r == 1, (s // _GRID) % _GRID, s % _GRID))
    mesh_ph = mesh.astype(jnp.float32) + jnp.float32(0.5)
    px_t_ref[...] = mesh_ph * ge * jnp.float32(2.0) + mn


def _sc_knn_body(xh, yh, zh, ph, out_h, xv, yv, zv, pv, ov):
    f32 = jnp.float32
    i32 = jnp.int32
    u32 = jnp.uint32
    wid = lax.axis_index("c") * 16 + lax.axis_index("s")
    base = wid * _CH
    pltpu.sync_copy(xh.at[pl.ds(base, _CH)], xv)
    pltpu.sync_copy(yh.at[pl.ds(base, _CH)], yv)
    pltpu.sync_copy(zh.at[pl.ds(base, _CH)], zv)
    pltpu.sync_copy(ph, pv)
    pvec = pv[...]
    mnx, mny, mnz = pvec[0], pvec[1], pvec[2]
    gex, gey, gez = pvec[3], pvec[4], pvec[5]
    inv_sx, inv_sy, inv_sz = pvec[6], pvec[7], pvec[8]
    inf_f = f32(jnp.inf)

    def center(idx_f, ge, mn):
        return (idx_f + f32(0.5)) * ge * f32(2.0) + mn

    def insert(best, v):
        for s in range(_A - 1):
            lo = jnp.minimum(best[s], v)
            v = jnp.maximum(best[s], v)
            best[s] = lo
        best[_A - 1] = jnp.minimum(best[_A - 1], v)
        return best

    def group_body(g, carry):
        g16 = g * 16
        xg = xv[pl.ds(g16, 16)]
        yg = yv[pl.ds(g16, 16)]
        zg = zv[pl.ds(g16, 16)]
        bx = jnp.clip(((xg - mnx) * inv_sx - f32(0.5)).astype(i32) - 1, 0, 4)
        by = jnp.clip(((yg - mny) * inv_sy - f32(0.5)).astype(i32) - 1, 0, 4)
        bz = jnp.clip(((zg - mnz) * inv_sz - f32(0.5)).astype(i32) - 1, 0, 4)
        ptsq = xg * xg + yg * yg + zg * zg
        xx2 = xg + xg
        yy2 = yg + yg
        zz2 = zg + zg
        ax, ay, az, gxv, gyv, gzv = [], [], [], [], [], []
        for o in range(4):
            cx = center((bx + o).astype(f32), gex, mnx)
            cy = center((by + o).astype(f32), gey, mny)
            cz = center((bz + o).astype(f32), gez, mnz)
            ax.append(cx * (cx - xx2) + ptsq)
            ay.append(cy * (cy - yy2))
            az.append(cz * (cz - zz2))
            gxv.append((bx + o) * 64)
            gyv.append((by + o) * 8)
            gzv.append(bz + o)
        best = [jnp.full((16,), u32(0xFFFFFFFF)) for _ in range(_A)]
        for ox in range(4):
            for oy in range(4):
                axy = ax[ox] + ay[oy]
                gxy = gxv[ox] + gyv[oy]
                for oz in range(4):
                    d2 = jnp.maximum(axy + az[oz], f32(0.0))
                    gi = gxy + gzv[oz]
                    key = ((lax.bitcast_convert_type(d2, u32) & u32(0xFFFFFE00))
                           | lax.bitcast_convert_type(gi, u32))
                    best = insert(best, key)
        def sqd(pg, b, ge, mn):
            dd = pg - center(b.astype(f32), ge, mn)
            return dd * dd

        e = jnp.minimum(
            jnp.minimum(
                jnp.minimum(jnp.where(bx > 0, sqd(xg, bx - 1, gex, mnx), inf_f),
                            jnp.where(bx < 4, sqd(xg, bx + 4, gex, mnx), inf_f)),
                jnp.minimum(jnp.where(by > 0, sqd(yg, by - 1, gey, mny), inf_f),
                            jnp.where(by < 4, sqd(yg, by + 4, gey, mny), inf_f))),
            jnp.minimum(jnp.where(bz > 0, sqd(zg, bz - 1, gez, mnz), inf_f),
                        jnp.where(bz < 4, sqd(zg, bz + 4, gez, mnz), inf_f)))
        ekey = lax.bitcast_convert_type(e, u32) & u32(0xFFFFFE00)
        safe = jnp.all(best[_A - 1] < ekey)

        def full_scan(args):
            xg, yg, zg, ptsq = args
            xx2 = xg + xg
            yy2 = yg + yg
            zz2 = zg + zg
            best = [jnp.full((16,), u32(0xFFFFFFFF)) for _ in range(_A)]

            def fb(r, best):
                best = list(best)
                for t in range(16):
                    s = r * 16 + t
                    cx = center((s // 64).astype(f32), gex, mnx)
                    cy = center(((s // 8) % 8).astype(f32), gey, mny)
                    cz = center((s % 8).astype(f32), gez, mnz)
                    d2 = jnp.maximum(((cx * cx + cy * cy + cz * cz) + ptsq)
                                     - (cx * xx2 + cy * yy2 + cz * zz2), f32(0.0))
                    key = ((lax.bitcast_convert_type(d2, u32) & u32(0xFFFFFE00))
                           | u32(s))
                    best = insert(best, key)
                return tuple(best)

            return lax.fori_loop(0, 32, fb, tuple(best))

        best = lax.cond(safe, lambda a: tuple(best), full_scan, (xg, yg, zg, ptsq))
        for k in range(_A):
            ov[pl.ds(k * _CH + g16, 16)] = lax.bitcast_convert_type(
                best[k] & u32(_S - 1), i32)
        return carry

    lax.fori_loop(0, _CH // 16, group_body, 0)
    for k in range(_A):
        pltpu.sync_copy(ov.at[pl.ds(k * _CH, _CH)], out_h.at[pl.ds(k * _PPAD + base, _CH)])


def kernel(point_pos):
    p = point_pos.shape[0]
    pts = jnp.pad(point_pos, ((0, _PPAD - p), (0, 0)), mode="edge")
    pts_t = pts.T

    px_t, mn, mx = pl.pallas_call(
        _grid_init_kernel,
        out_shape=(
            jax.ShapeDtypeStruct((_DIM, _S), jnp.float32),
            jax.ShapeDtypeStruct((_DIM, 1), jnp.float32),
            jax.ShapeDtypeStruct((_DIM, 1), jnp.float32),
        ),
    )(pts_t)

    mn1 = mn[:, 0]
    ge = (mx[:, 0] - mn1) / jnp.float32(_GRID) * jnp.float32(0.5)
    inv_step = jnp.float32(1.0) / (ge * jnp.float32(2.0))
    params = jnp.concatenate([mn1, ge, inv_step, jnp.zeros((7,), jnp.float32)])

    mesh = plsc.VectorSubcoreMesh(core_axis_name="c", subcore_axis_name="s")
    idx = pl.kernel(
        _sc_knn_body,
        out_type=jax.ShapeDtypeStruct((_PPAD * _A,), jnp.int32),
        mesh=mesh,
        compiler_params=pltpu.CompilerParams(needs_layout_passes=False),
        scratch_types=[
            pltpu.VMEM((_CH,), jnp.float32),
            pltpu.VMEM((_CH,), jnp.float32),
            pltpu.VMEM((_CH,), jnp.float32),
            pltpu.VMEM((16,), jnp.float32),
            pltpu.VMEM((_CH * _A,), jnp.int32),
        ],
    )(pts_t[0], pts_t[1], pts_t[2], params)
    idx = idx.reshape(_A, _PPAD).T

    px_pos = px_t.T
    pt_ids = jnp.repeat(jnp.arange(p, dtype=jnp.int32), _A)
    px_ids = idx[:p].reshape(-1)
    assoc = jnp.stack([pt_ids, px_ids], axis=-1)
    return px_pos, assoc
